# baseline (speedup 1.0000x reference)
import dataclasses

import jax
import jax.numpy as jnp
from jax import lax
from jax.experimental import pallas as pl
from jax.experimental.pallas import tpu as pltpu
from jax.experimental.pallas import tpu_sc as plsc

N_FACTORS = 32
N_CLASSES = 5
BATCH = 16384
NUM_WORKERS = 32
ROWS_PER_WORKER = BATCH // NUM_WORKERS
GATHER_CHUNK = 128
NUM_GATHERS = ROWS_PER_WORKER // GATHER_CHUNK
LANES = 16
CHUNKS_PER_GROUP = 4
GROUP_ROWS = LANES * CHUNKS_PER_GROUP
NUM_GROUPS = ROWS_PER_WORKER // GROUP_ROWS


def _sc_body(user_ref, item_ref, ut_ref, it_ref, w_ref, b_ref, out_ref,
             uidx, vidx, u_rows, v_rows, w_vmem, b_vmem, wb_vmem,
             out_buf, sem):
    core = lax.axis_index("core")
    subcore = lax.axis_index("subcore")
    wid = subcore * 2 + core
    base = wid * ROWS_PER_WORKER

    pltpu.sync_copy(user_ref.at[pl.ds(base, ROWS_PER_WORKER)], uidx)
    pltpu.sync_copy(item_ref.at[pl.ds(base, ROWS_PER_WORKER)], vidx)
    pltpu.sync_copy(w_ref, w_vmem)
    pltpu.sync_copy(b_ref, b_vmem)

    copies = []
    for j in range(NUM_GATHERS):
        sl = pl.ds(j * GATHER_CHUNK, GATHER_CHUNK)
        copies.append(pltpu.async_copy(ut_ref.at[uidx.at[sl]], u_rows.at[sl], sem))
        copies.append(pltpu.async_copy(it_ref.at[vidx.at[sl]], v_rows.at[sl], sem))

    @pl.loop(0, N_FACTORS * N_CLASSES)
    def _(k):
        kv = jnp.full((LANES,), 0, jnp.int32) + k
        wb_vmem[pl.ds(k * LANES, LANES)] = plsc.load_gather(w_vmem, [kv])

    bias = [plsc.load_gather(b_vmem, [jnp.full((LANES,), 8 + c, jnp.int32)])
            for c in range(N_CLASSES)]

    for c in copies:
        c.wait()

    iota16 = lax.iota(jnp.int32, LANES)

    @pl.loop(0, NUM_GROUPS)
    def _(group):
        rows = [iota16 + (group * GROUP_ROWS + r * LANES)
                for r in range(CHUNKS_PER_GROUP)]
        accs = [[bias[c] for c in range(N_CLASSES)]
                for _ in range(CHUNKS_PER_GROUP)]
        for f in range(N_FACTORS):
            col = jnp.full((LANES,), f, jnp.int32)
            wv = [wb_vmem[pl.ds((f * N_CLASSES + c) * LANES, LANES)]
                  for c in range(N_CLASSES)]
            for r in range(CHUNKS_PER_GROUP):
                uf = plsc.load_gather(u_rows, [rows[r], col])
                vf = plsc.load_gather(v_rows, [rows[r], col])
                feat = uf * vf
                for c in range(N_CLASSES):
                    accs[r][c] = accs[r][c] + feat * wv[c]
        for r in range(CHUNKS_PER_GROUP):
            for c in range(N_CLASSES):
                out_buf[c, pl.ds(group * GROUP_ROWS + r * LANES, LANES)] = accs[r][c]

    for c in range(N_CLASSES):
        pltpu.sync_copy(out_buf.at[pl.ds(c, 1), :],
                        out_ref.at[pl.ds(c, 1), pl.ds(base, ROWS_PER_WORKER)])


def kernel(user, item, user_table, item_table, W, b):
    mesh = plsc.VectorSubcoreMesh(core_axis_name="core",
                                  subcore_axis_name="subcore")
    cp = pltpu.CompilerParams(use_tc_tiling_on_sc=False)
    if "needs_layout_passes" in pltpu.CompilerParams.__dataclass_fields__:
        cp = dataclasses.replace(cp, needs_layout_passes=False)
    k = pl.kernel(
        _sc_body,
        out_type=jax.ShapeDtypeStruct((N_CLASSES, BATCH), jnp.float32),
        mesh=mesh,
        compiler_params=cp,
        scratch_types=[
            pltpu.VMEM((ROWS_PER_WORKER,), jnp.int32),
            pltpu.VMEM((ROWS_PER_WORKER,), jnp.int32),
            pltpu.VMEM((ROWS_PER_WORKER, N_FACTORS), jnp.float32),
            pltpu.VMEM((ROWS_PER_WORKER, N_FACTORS), jnp.float32),
            pltpu.VMEM((N_FACTORS * N_CLASSES,), jnp.float32),
            pltpu.VMEM((LANES,), jnp.float32),
            pltpu.VMEM((N_FACTORS * N_CLASSES * LANES,), jnp.float32),
            pltpu.VMEM((N_CLASSES, ROWS_PER_WORKER), jnp.float32),
            pltpu.SemaphoreType.DMA,
        ],
    )
    b_pad = jnp.zeros((LANES,), jnp.float32).at[8:8 + N_CLASSES].set(b)
    out = k(user.astype(jnp.int32), item.astype(jnp.int32),
            user_table, item_table, W.reshape(-1), b_pad)
    return out.T
